# Initial kernel scaffold; baseline (speedup 1.0000x reference)
import jax
import jax.numpy as jnp
from jax.experimental import pallas as pl


def kernel(node_features, edge_index, edge_sh, edge_radial_emb, Wq, Wk1, bk1, Wk2, bk2, Wv1, bv1, Wv2, bv2, Wdot, Wout, Wf1, Wf2):
    raise NotImplementedError("write your pallas kernel here")


# speedup vs baseline: 29.8150x; 29.8150x over previous
import functools

import numpy as np
import jax
import jax.numpy as jnp
from jax import lax
from jax.experimental import pallas as pl
from jax.experimental.pallas import tpu as pltpu
from jax.experimental.pallas import tpu_sc as plsc

N = 10000
E = 160000
D = 16
H = 4
DH = D // H
NB = 16
HID = 64

NW = 32
CHUNK = 128
EPAD = 163840
EPW = EPAD // NW
NITER = EPW // CHUNK
NPAD = 10240
NPS = NPAD // 16
PW = 32

TE = 2048
TN = 1024

@functools.lru_cache(maxsize=None)
def _build_sc_kernels():
    mesh = plsc.VectorSubcoreMesh(core_axis_name="c", subcore_axis_name="s")

    @functools.partial(
        pl.kernel,
        out_type=(jax.ShapeDtypeStruct((EPAD, D), jnp.float32),
                  jax.ShapeDtypeStruct((EPAD, D), jnp.float32)),
        mesh=mesh,
        scratch_types=[
            pltpu.VMEM((CHUNK,), jnp.int32),
            pltpu.VMEM((CHUNK,), jnp.int32),
            pltpu.VMEM((CHUNK, D), jnp.float32),
            pltpu.VMEM((CHUNK, D), jnp.float32),
            pltpu.SemaphoreType.DMA,
            pltpu.SemaphoreType.DMA,
        ],
        compiler_params=pltpu.CompilerParams(use_tc_tiling_on_sc=False),
    )
    def sc_gather(nf_hbm, src_hbm, dst_hbm, xs_hbm, xd_hbm,
                  idxa, idxb, rowsa, rowsb, sema, semb):
        c = lax.axis_index("c")
        s = lax.axis_index("s")
        wid = s * 2 + c
        base = wid * EPW

        def body(i, carry):
            off = base + i * CHUNK
            pltpu.sync_copy(src_hbm.at[pl.ds(off, CHUNK)], idxa)
            pltpu.sync_copy(dst_hbm.at[pl.ds(off, CHUNK)], idxb)
            cpa = pltpu.async_copy(nf_hbm.at[idxa], rowsa, sema)
            cpb = pltpu.async_copy(nf_hbm.at[idxb], rowsb, semb)
            cpa.wait()
            cpb.wait()
            pltpu.sync_copy(rowsa, xs_hbm.at[pl.ds(off, CHUNK)])
            pltpu.sync_copy(rowsb, xd_hbm.at[pl.ds(off, CHUNK)])
            return carry

        lax.fori_loop(0, NITER, body, 0)

    @functools.partial(
        pl.kernel,
        out_type=jax.ShapeDtypeStruct((2 * NPAD, PW), jnp.float32),
        mesh=mesh,
        scratch_types=[
            pltpu.VMEM((CHUNK,), jnp.int32),
            pltpu.VMEM((CHUNK, PW), jnp.float32),
            pltpu.VMEM((NPS, PW), jnp.float32),
            pltpu.VMEM_SHARED((NPAD, PW), jnp.float32),
        ],
        compiler_params=pltpu.CompilerParams(use_tc_tiling_on_sc=False),
    )
    def sc_aggregate(payload_hbm, dst_hbm, zeros_hbm, out_hbm,
                     idxv, rowsv, outv, acc_sh):
        c = lax.axis_index("c")
        s = lax.axis_index("s")
        pltpu.sync_copy(zeros_hbm.at[pl.ds(s * NPS, NPS)],
                        acc_sh.at[pl.ds(s * NPS, NPS)])
        plsc.subcore_barrier()
        wid = s * 2 + c
        base = wid * EPW

        def body(i, carry):
            off = base + i * CHUNK
            pltpu.sync_copy(dst_hbm.at[pl.ds(off, CHUNK)], idxv)
            pltpu.sync_copy(payload_hbm.at[pl.ds(off, CHUNK)], rowsv)
            pltpu.sync_copy(rowsv, acc_sh.at[idxv], add=True)
            return carry

        lax.fori_loop(0, NITER, body, 0)
        plsc.subcore_barrier()
        pltpu.sync_copy(acc_sh.at[pl.ds(s * NPS, NPS)], outv)
        pltpu.sync_copy(outv, out_hbm.at[pl.ds(c * NPAD + s * NPS, NPS)])

    return sc_gather, sc_aggregate


def _silu(x):
    return x * (1.0 / (1.0 + jnp.exp(-x)))


def _edge_body(r_ref, xs_ref, xd_ref, sh_ref,
               wk1_ref, bk1_ref, wk2_ref, bk2_ref,
               wv1_ref, bv1_ref, wv2_ref, bv2_ref,
               wqd_ref, rrep_ref, ssum_ref, shf_ref, out_ref):
    r = r_ref[...]
    hk = _silu(r @ wk1_ref[...] + bk1_ref[...])
    kw = hk @ wk2_ref[...] + bk2_ref[...]
    hv = _silu(r @ wv1_ref[...] + bv1_ref[...])
    vw = hv @ wv2_ref[...] + bv2_ref[...]
    xr = xs_ref[...] @ rrep_ref[...]
    kraw = (xr * kw) @ ssum_ref[...]
    vraw = (xr * vw) @ ssum_ref[...]
    qw = xd_ref[...] @ wqd_ref[...]
    sh = sh_ref[...]
    lg = ((qw * kraw) @ shf_ref[...]) * sh
    ex = jnp.exp(lg)
    vact = vraw * (sh * 0.25)
    exv = ex * vact
    out_ref[...] = jnp.concatenate([exv, ex], axis=1)


def _edge_stage(r, xs, xd, sh, wk1, bk1, wk2, bk2, wv1, bv1, wv2, bv2,
                wqd, rrep, ssum, shf):
    edge_spec = lambda w: pl.BlockSpec((TE, w), lambda i: (i, 0))
    full_spec = lambda a, b: pl.BlockSpec((a, b), lambda i: (0, 0))
    return pl.pallas_call(
        _edge_body,
        grid=(EPAD // TE,),
        in_specs=[
            edge_spec(NB), edge_spec(D), edge_spec(D), edge_spec(1),
            full_spec(NB, HID), full_spec(1, HID),
            full_spec(HID, D * D), full_spec(1, D * D),
            full_spec(NB, HID), full_spec(1, HID),
            full_spec(HID, D * D), full_spec(1, D * D),
            full_spec(D, D), full_spec(D, D * D),
            full_spec(D * D, D), full_spec(D, D),
        ],
        out_specs=pl.BlockSpec((TE, PW), lambda i: (i, 0)),
        out_shape=jax.ShapeDtypeStruct((EPAD, PW), jnp.float32),
    )(r, xs, xd, sh, wk1, bk1, wk2, bk2, wv1, bv1, wv2, bv2,
      wqd, rrep, ssum, shf)


def _final_body(nf_ref, p0_ref, p1_ref, wout_ref, wf1_ref, wf2_ref, out_ref):
    acc = p0_ref[...] + p1_ref[...]
    num = acc[:, :D]
    den = acc[:, D:]
    out_h = num / (den + 1e-9)
    node = nf_ref[...] + out_h @ wout_ref[...]
    hq = _silu(node @ wf1_ref[...])
    out_ref[...] = node + hq @ wf2_ref[...]


def _final_stage(nf_pad, parts, wout4, wf1s, wf2s):
    return pl.pallas_call(
        _final_body,
        grid=(NPAD // TN,),
        in_specs=[
            pl.BlockSpec((TN, D), lambda i: (i, 0)),
            pl.BlockSpec((TN, PW), lambda i: (i, 0)),
            pl.BlockSpec((TN, PW), lambda i: (i + NPAD // TN, 0)),
            pl.BlockSpec((D, D), lambda i: (0, 0)),
            pl.BlockSpec((D, 2 * D), lambda i: (0, 0)),
            pl.BlockSpec((2 * D, D), lambda i: (0, 0)),
        ],
        out_specs=pl.BlockSpec((TN, D), lambda i: (i, 0)),
        out_shape=jax.ShapeDtypeStruct((NPAD, D), jnp.float32),
    )(nf_pad, parts, parts, wout4, wf1s, wf2s)


_RREP = np.kron(np.eye(D), np.ones((1, D))).astype(np.float32)
_SSUM = np.kron(np.ones((D, 1)), np.eye(D)).astype(np.float32)
_SHF = np.kron(np.eye(H), np.ones((DH, DH))).astype(np.float32)


def kernel(node_features, edge_index, edge_sh, edge_radial_emb,
           Wq, Wk1, bk1, Wk2, bk2, Wv1, bv1, Wv2, bv2, Wdot, Wout, Wf1, Wf2):
    f32 = jnp.float32
    src = jnp.concatenate(
        [edge_index[0].astype(jnp.int32),
         jnp.zeros((EPAD - E,), jnp.int32)])
    dst = jnp.concatenate(
        [edge_index[1].astype(jnp.int32),
         jnp.full((EPAD - E,), NPAD - 1, jnp.int32)])
    r = jnp.concatenate(
        [edge_radial_emb.astype(f32), jnp.zeros((EPAD - E, NB), f32)])
    sh = jnp.concatenate(
        [edge_sh.astype(f32), jnp.zeros((EPAD - E, 1), f32)])
    nf_pad = jnp.concatenate(
        [node_features.astype(f32), jnp.zeros((NPAD - N, D), f32)])

    wdot_bd = jnp.kron(jnp.eye(H, dtype=f32), Wdot.astype(f32))
    wqd = (Wq.astype(f32) @ wdot_bd) * (1.0 / 128.0)
    wout4 = Wout.astype(f32) * 0.25
    wf1s = Wf1.astype(f32) * 0.25
    wf2s = Wf2.astype(f32) * (1.0 / np.sqrt(2 * D))

    sc_gather, sc_aggregate = _build_sc_kernels()
    xs, xd = sc_gather(nf_pad, src, dst)
    payload = _edge_stage(
        r, xs, xd, sh,
        Wk1.astype(f32), bk1.astype(f32).reshape(1, HID),
        Wk2.astype(f32), bk2.astype(f32).reshape(1, D * D),
        Wv1.astype(f32), bv1.astype(f32).reshape(1, HID),
        Wv2.astype(f32), bv2.astype(f32).reshape(1, D * D),
        wqd, jnp.asarray(_RREP), jnp.asarray(_SSUM), jnp.asarray(_SHF))
    zeros = jnp.zeros((NPAD, PW), f32)
    parts = sc_aggregate(payload, dst, zeros)
    out = _final_stage(nf_pad, parts, wout4, wf1s, wf2s)
    return out[:N]


# speedup vs baseline: 32.1228x; 1.0774x over previous
import functools

import numpy as np
import jax
import jax.numpy as jnp
from jax import lax
from jax.experimental import pallas as pl
from jax.experimental.pallas import tpu as pltpu
from jax.experimental.pallas import tpu_sc as plsc

N = 10000
E = 160000
D = 16
H = 4
DH = D // H
NB = 16
HID = 64

NW = 32
CHUNK = 128
EPAD = 163840
EPW = EPAD // NW
NITER = EPW // CHUNK
SLAB = 8
NPAD = 10240
NPS = NPAD // 16
PW = 32

TE = 2048
TN = 1024

@functools.lru_cache(maxsize=None)
def _build_sc_kernels():
    mesh = plsc.VectorSubcoreMesh(core_axis_name="c", subcore_axis_name="s")

    @functools.partial(
        pl.kernel,
        out_type=(jax.ShapeDtypeStruct((EPAD, D), jnp.float32),
                  jax.ShapeDtypeStruct((EPAD, D), jnp.float32)),
        mesh=mesh,
        scratch_types=[
            pltpu.VMEM((NITER, CHUNK), jnp.int32),
            pltpu.VMEM((NITER, CHUNK), jnp.int32),
            pltpu.VMEM((EPW, D), jnp.float32),
            pltpu.SemaphoreType.DMA,
        ],
        compiler_params=pltpu.CompilerParams(use_tc_tiling_on_sc=False),
    )
    def sc_gather(nf_hbm, src2_hbm, dst2_hbm, xs_hbm, xd_hbm,
                  idxs2, idxd2, rows, sem):
        c = lax.axis_index("c")
        s = lax.axis_index("s")
        wid = s * 2 + c
        pltpu.sync_copy(src2_hbm.at[pl.ds(wid * NITER, NITER)], idxs2)
        pltpu.sync_copy(dst2_hbm.at[pl.ds(wid * NITER, NITER)], idxd2)
        ebase = wid * EPW

        def one_pass(idx2, out_hbm):
            def body(b, carry):
                cps = []
                for j in range(SLAB):
                    k = b * SLAB + j
                    cps.append(pltpu.async_copy(
                        nf_hbm.at[idx2.at[k]],
                        rows.at[pl.ds(k * CHUNK, CHUNK)], sem))
                for cp in cps:
                    cp.wait()
                return carry

            lax.fori_loop(0, NITER // SLAB, body, 0)
            pltpu.sync_copy(rows, out_hbm.at[pl.ds(ebase, EPW)])

        one_pass(idxs2, xs_hbm)
        one_pass(idxd2, xd_hbm)

    @functools.partial(
        pl.kernel,
        out_type=jax.ShapeDtypeStruct((2 * NPAD, PW), jnp.float32),
        mesh=mesh,
        scratch_types=[
            pltpu.VMEM((NITER, CHUNK), jnp.int32),
            pltpu.VMEM((SLAB * CHUNK, PW), jnp.float32),
            pltpu.VMEM((NPS, PW), jnp.float32),
            pltpu.VMEM_SHARED((NPAD, PW), jnp.float32),
            pltpu.SemaphoreType.DMA,
        ],
        compiler_params=pltpu.CompilerParams(use_tc_tiling_on_sc=False),
    )
    def sc_aggregate(payload_hbm, dst2_hbm, zeros_hbm, out_hbm,
                     idxd2, pbuf, outv, acc_sh, sem):
        c = lax.axis_index("c")
        s = lax.axis_index("s")
        pltpu.sync_copy(zeros_hbm.at[pl.ds(s * NPS, NPS)],
                        acc_sh.at[pl.ds(s * NPS, NPS)])
        plsc.subcore_barrier()
        wid = s * 2 + c
        pltpu.sync_copy(dst2_hbm.at[pl.ds(wid * NITER, NITER)], idxd2)
        ebase = wid * EPW

        def body(b, carry):
            pltpu.sync_copy(
                payload_hbm.at[pl.ds(ebase + b * SLAB * CHUNK, SLAB * CHUNK)],
                pbuf)
            cps = []
            for j in range(SLAB):
                k = b * SLAB + j
                cps.append(pltpu.async_copy(
                    pbuf.at[pl.ds(j * CHUNK, CHUNK)],
                    acc_sh.at[idxd2.at[k]], sem, add=True))
            for cp in cps:
                cp.wait()
            return carry

        lax.fori_loop(0, NITER // SLAB, body, 0)
        plsc.subcore_barrier()
        pltpu.sync_copy(acc_sh.at[pl.ds(s * NPS, NPS)], outv)
        pltpu.sync_copy(outv, out_hbm.at[pl.ds(c * NPAD + s * NPS, NPS)])

    return sc_gather, sc_aggregate


def _silu(x):
    return x * (1.0 / (1.0 + jnp.exp(-x)))


def _edge_body(r_ref, xs_ref, xd_ref, sh_ref,
               wk1_ref, bk1_ref, wk2_ref, bk2_ref,
               wv1_ref, bv1_ref, wv2_ref, bv2_ref,
               wqd_ref, rrep_ref, ssum_ref, shf_ref, out_ref):
    r = r_ref[...]
    hk = _silu(r @ wk1_ref[...] + bk1_ref[...])
    kw = hk @ wk2_ref[...] + bk2_ref[...]
    hv = _silu(r @ wv1_ref[...] + bv1_ref[...])
    vw = hv @ wv2_ref[...] + bv2_ref[...]
    xr = xs_ref[...] @ rrep_ref[...]
    kraw = (xr * kw) @ ssum_ref[...]
    vraw = (xr * vw) @ ssum_ref[...]
    qw = xd_ref[...] @ wqd_ref[...]
    sh = sh_ref[...]
    lg = ((qw * kraw) @ shf_ref[...]) * sh
    ex = jnp.exp(lg)
    vact = vraw * (sh * 0.25)
    exv = ex * vact
    out_ref[...] = jnp.concatenate([exv, ex], axis=1)


def _edge_stage(r, xs, xd, sh, wk1, bk1, wk2, bk2, wv1, bv1, wv2, bv2,
                wqd, rrep, ssum, shf):
    edge_spec = lambda w: pl.BlockSpec((TE, w), lambda i: (i, 0))
    full_spec = lambda a, b: pl.BlockSpec((a, b), lambda i: (0, 0))
    return pl.pallas_call(
        _edge_body,
        grid=(EPAD // TE,),
        in_specs=[
            edge_spec(NB), edge_spec(D), edge_spec(D), edge_spec(1),
            full_spec(NB, HID), full_spec(1, HID),
            full_spec(HID, D * D), full_spec(1, D * D),
            full_spec(NB, HID), full_spec(1, HID),
            full_spec(HID, D * D), full_spec(1, D * D),
            full_spec(D, D), full_spec(D, D * D),
            full_spec(D * D, D), full_spec(D, D),
        ],
        out_specs=pl.BlockSpec((TE, PW), lambda i: (i, 0)),
        out_shape=jax.ShapeDtypeStruct((EPAD, PW), jnp.float32),
    )(r, xs, xd, sh, wk1, bk1, wk2, bk2, wv1, bv1, wv2, bv2,
      wqd, rrep, ssum, shf)


def _final_body(nf_ref, p0_ref, p1_ref, wout_ref, wf1_ref, wf2_ref, out_ref):
    acc = p0_ref[...] + p1_ref[...]
    num = acc[:, :D]
    den = acc[:, D:]
    out_h = num / (den + 1e-9)
    node = nf_ref[...] + out_h @ wout_ref[...]
    hq = _silu(node @ wf1_ref[...])
    out_ref[...] = node + hq @ wf2_ref[...]


def _final_stage(nf_pad, parts, wout4, wf1s, wf2s):
    return pl.pallas_call(
        _final_body,
        grid=(NPAD // TN,),
        in_specs=[
            pl.BlockSpec((TN, D), lambda i: (i, 0)),
            pl.BlockSpec((TN, PW), lambda i: (i, 0)),
            pl.BlockSpec((TN, PW), lambda i: (i + NPAD // TN, 0)),
            pl.BlockSpec((D, D), lambda i: (0, 0)),
            pl.BlockSpec((D, 2 * D), lambda i: (0, 0)),
            pl.BlockSpec((2 * D, D), lambda i: (0, 0)),
        ],
        out_specs=pl.BlockSpec((TN, D), lambda i: (i, 0)),
        out_shape=jax.ShapeDtypeStruct((NPAD, D), jnp.float32),
    )(nf_pad, parts, parts, wout4, wf1s, wf2s)


_RREP = np.kron(np.eye(D), np.ones((1, D))).astype(np.float32)
_SSUM = np.kron(np.ones((D, 1)), np.eye(D)).astype(np.float32)
_SHF = np.kron(np.eye(H), np.ones((DH, DH))).astype(np.float32)


def kernel(node_features, edge_index, edge_sh, edge_radial_emb,
           Wq, Wk1, bk1, Wk2, bk2, Wv1, bv1, Wv2, bv2, Wdot, Wout, Wf1, Wf2):
    f32 = jnp.float32
    src = jnp.concatenate(
        [edge_index[0].astype(jnp.int32),
         jnp.zeros((EPAD - E,), jnp.int32)])
    dst = jnp.concatenate(
        [edge_index[1].astype(jnp.int32),
         jnp.full((EPAD - E,), NPAD - 1, jnp.int32)])
    r = jnp.concatenate(
        [edge_radial_emb.astype(f32), jnp.zeros((EPAD - E, NB), f32)])
    sh = jnp.concatenate(
        [edge_sh.astype(f32), jnp.zeros((EPAD - E, 1), f32)])
    nf_pad = jnp.concatenate(
        [node_features.astype(f32), jnp.zeros((NPAD - N, D), f32)])

    wdot_bd = jnp.kron(jnp.eye(H, dtype=f32), Wdot.astype(f32))
    wqd = (Wq.astype(f32) @ wdot_bd) * (1.0 / 128.0)
    wout4 = Wout.astype(f32) * 0.25
    wf1s = Wf1.astype(f32) * 0.25
    wf2s = Wf2.astype(f32) * (1.0 / np.sqrt(2 * D))

    sc_gather, sc_aggregate = _build_sc_kernels()
    src2 = src.reshape(EPAD // CHUNK, CHUNK)
    dst2 = dst.reshape(EPAD // CHUNK, CHUNK)
    xs, xd = sc_gather(nf_pad, src2, dst2)
    payload = _edge_stage(
        r, xs, xd, sh,
        Wk1.astype(f32), bk1.astype(f32).reshape(1, HID),
        Wk2.astype(f32), bk2.astype(f32).reshape(1, D * D),
        Wv1.astype(f32), bv1.astype(f32).reshape(1, HID),
        Wv2.astype(f32), bv2.astype(f32).reshape(1, D * D),
        wqd, jnp.asarray(_RREP), jnp.asarray(_SSUM), jnp.asarray(_SHF))
    zeros = jnp.zeros((NPAD, PW), f32)
    parts = sc_aggregate(payload, dst2, zeros)
    out = _final_stage(nf_pad, parts, wout4, wf1s, wf2s)
    return out[:N]


# speedup vs baseline: 34.3765x; 1.0702x over previous
import functools

import numpy as np
import jax
import jax.numpy as jnp
from jax import lax
from jax.experimental import pallas as pl
from jax.experimental.pallas import tpu as pltpu
from jax.experimental.pallas import tpu_sc as plsc

N = 10000
E = 160000
D = 16
H = 4
DH = D // H
NB = 16
HID = 64

NW = 32
CHUNK = 128
EPAD = 163840
EPW = EPAD // NW
NITER = EPW // CHUNK
SLAB = 16
NPAD = 10240
NPS = NPAD // 16
PW = 32

TE = 4096
TN = 1024

@functools.lru_cache(maxsize=None)
def _build_sc_kernels():
    mesh = plsc.VectorSubcoreMesh(core_axis_name="c", subcore_axis_name="s")

    @functools.partial(
        pl.kernel,
        out_type=(jax.ShapeDtypeStruct((EPAD, D), jnp.float32),
                  jax.ShapeDtypeStruct((EPAD, D), jnp.float32)),
        mesh=mesh,
        scratch_types=[
            pltpu.VMEM((NITER, CHUNK), jnp.int32),
            pltpu.VMEM((NITER, CHUNK), jnp.int32),
            pltpu.VMEM((EPW, D), jnp.float32),
            pltpu.SemaphoreType.DMA,
        ],
        compiler_params=pltpu.CompilerParams(use_tc_tiling_on_sc=False),
    )
    def sc_gather(nf_hbm, src2_hbm, dst2_hbm, xs_hbm, xd_hbm,
                  idxs2, idxd2, rows, sem):
        c = lax.axis_index("c")
        s = lax.axis_index("s")
        wid = s * 2 + c
        pltpu.sync_copy(src2_hbm.at[pl.ds(wid * NITER, NITER)], idxs2)
        pltpu.sync_copy(dst2_hbm.at[pl.ds(wid * NITER, NITER)], idxd2)
        ebase = wid * EPW

        def one_pass(idx2, out_hbm):
            def body(b, carry):
                cps = []
                for j in range(SLAB):
                    k = b * SLAB + j
                    cps.append(pltpu.async_copy(
                        nf_hbm.at[idx2.at[k]],
                        rows.at[pl.ds(k * CHUNK, CHUNK)], sem))
                for cp in cps:
                    cp.wait()
                return carry

            lax.fori_loop(0, NITER // SLAB, body, 0)
            pltpu.sync_copy(rows, out_hbm.at[pl.ds(ebase, EPW)])

        one_pass(idxs2, xs_hbm)
        one_pass(idxd2, xd_hbm)

    @functools.partial(
        pl.kernel,
        out_type=jax.ShapeDtypeStruct((2 * NPAD, PW), jnp.float32),
        mesh=mesh,
        scratch_types=[
            pltpu.VMEM((NITER, CHUNK), jnp.int32),
            pltpu.VMEM((SLAB * CHUNK, PW), jnp.float32),
            pltpu.VMEM((NPS, PW), jnp.float32),
            pltpu.VMEM_SHARED((NPAD, PW), jnp.float32),
            pltpu.SemaphoreType.DMA,
        ],
        compiler_params=pltpu.CompilerParams(use_tc_tiling_on_sc=False),
    )
    def sc_aggregate(payload_hbm, dst2_hbm, zeros_hbm, out_hbm,
                     idxd2, pbuf, outv, acc_sh, sem):
        c = lax.axis_index("c")
        s = lax.axis_index("s")
        pltpu.sync_copy(zeros_hbm.at[pl.ds(s * NPS, NPS)],
                        acc_sh.at[pl.ds(s * NPS, NPS)])
        plsc.subcore_barrier()
        wid = s * 2 + c
        pltpu.sync_copy(dst2_hbm.at[pl.ds(wid * NITER, NITER)], idxd2)
        ebase = wid * EPW

        def body(b, carry):
            pltpu.sync_copy(
                payload_hbm.at[pl.ds(ebase + b * SLAB * CHUNK, SLAB * CHUNK)],
                pbuf)
            cps = []
            for j in range(SLAB):
                k = b * SLAB + j
                cps.append(pltpu.async_copy(
                    pbuf.at[pl.ds(j * CHUNK, CHUNK)],
                    acc_sh.at[idxd2.at[k]], sem, add=True))
            for cp in cps:
                cp.wait()
            return carry

        lax.fori_loop(0, NITER // SLAB, body, 0)
        plsc.subcore_barrier()
        pltpu.sync_copy(acc_sh.at[pl.ds(s * NPS, NPS)], outv)
        pltpu.sync_copy(outv, out_hbm.at[pl.ds(c * NPAD + s * NPS, NPS)])

    return sc_gather, sc_aggregate


def _silu(x):
    return x * (1.0 / (1.0 + jnp.exp(-x)))


def _edge_body(r_ref, xs_ref, xd_ref, sh_ref,
               w1_ref, b1_ref, wk2_ref, bk2_ref, bv2_ref, wv2_ref,
               wqd_ref, rrep_ref, ssum_ref, shf_ref, out_ref):
    r = r_ref[...]
    bf = jnp.bfloat16
    f32 = jnp.float32
    h1 = _silu(r @ w1_ref[...] + b1_ref[...])
    hk = h1[:, :HID]
    hv = h1[:, HID:]
    kw = (jnp.dot(hk.astype(bf), wk2_ref[...].astype(bf),
                  preferred_element_type=f32) + bk2_ref[...]).astype(bf)
    vw = (jnp.dot(hv.astype(bf), wv2_ref[...].astype(bf),
                  preferred_element_type=f32) + bv2_ref[...]).astype(bf)
    xr = jnp.dot(xs_ref[...].astype(bf), rrep_ref[...].astype(bf),
                 preferred_element_type=f32).astype(bf)
    kraw = jnp.dot(xr * kw, ssum_ref[...].astype(bf),
                   preferred_element_type=f32)
    vraw = jnp.dot(xr * vw, ssum_ref[...].astype(bf),
                   preferred_element_type=f32)
    qw = xd_ref[...] @ wqd_ref[...]
    sh = sh_ref[...]
    lg = ((qw * kraw) @ shf_ref[...]) * sh
    ex = jnp.exp(lg)
    vact = vraw * (sh * 0.25)
    exv = ex * vact
    out_ref[...] = jnp.concatenate([exv, ex], axis=1)


def _edge_stage(r, xs, xd, sh, w1, b1, wk2, bk2, bv2, wv2,
                wqd, rrep, ssum, shf):
    edge_spec = lambda w: pl.BlockSpec((TE, w), lambda i: (i, 0))
    full_spec = lambda a, b: pl.BlockSpec((a, b), lambda i: (0, 0))
    return pl.pallas_call(
        _edge_body,
        grid=(EPAD // TE,),
        in_specs=[
            edge_spec(NB), edge_spec(D), edge_spec(D), edge_spec(1),
            full_spec(NB, 2 * HID), full_spec(1, 2 * HID),
            full_spec(HID, D * D), full_spec(1, D * D),
            full_spec(1, D * D), full_spec(HID, D * D),
            full_spec(D, D), full_spec(D, D * D),
            full_spec(D * D, D), full_spec(D, D),
        ],
        out_specs=pl.BlockSpec((TE, PW), lambda i: (i, 0)),
        out_shape=jax.ShapeDtypeStruct((EPAD, PW), jnp.float32),
    )(r, xs, xd, sh, w1, b1, wk2, bk2, bv2, wv2,
      wqd, rrep, ssum, shf)


def _final_body(nf_ref, p0_ref, p1_ref, wout_ref, wf1_ref, wf2_ref, out_ref):
    acc = p0_ref[...] + p1_ref[...]
    num = acc[:, :D]
    den = acc[:, D:]
    out_h = num / (den + 1e-9)
    node = nf_ref[...] + out_h @ wout_ref[...]
    hq = _silu(node @ wf1_ref[...])
    out_ref[...] = node + hq @ wf2_ref[...]


def _final_stage(nf_pad, parts, wout4, wf1s, wf2s):
    return pl.pallas_call(
        _final_body,
        grid=(NPAD // TN,),
        in_specs=[
            pl.BlockSpec((TN, D), lambda i: (i, 0)),
            pl.BlockSpec((TN, PW), lambda i: (i, 0)),
            pl.BlockSpec((TN, PW), lambda i: (i + NPAD // TN, 0)),
            pl.BlockSpec((D, D), lambda i: (0, 0)),
            pl.BlockSpec((D, 2 * D), lambda i: (0, 0)),
            pl.BlockSpec((2 * D, D), lambda i: (0, 0)),
        ],
        out_specs=pl.BlockSpec((TN, D), lambda i: (i, 0)),
        out_shape=jax.ShapeDtypeStruct((NPAD, D), jnp.float32),
    )(nf_pad, parts, parts, wout4, wf1s, wf2s)


_RREP = np.kron(np.eye(D), np.ones((1, D))).astype(np.float32)
_SSUM = np.kron(np.ones((D, 1)), np.eye(D)).astype(np.float32)
_SHF = np.kron(np.eye(H), np.ones((DH, DH))).astype(np.float32)


def kernel(node_features, edge_index, edge_sh, edge_radial_emb,
           Wq, Wk1, bk1, Wk2, bk2, Wv1, bv1, Wv2, bv2, Wdot, Wout, Wf1, Wf2):
    f32 = jnp.float32
    src = jnp.concatenate(
        [edge_index[0].astype(jnp.int32),
         jnp.zeros((EPAD - E,), jnp.int32)])
    dst = jnp.concatenate(
        [edge_index[1].astype(jnp.int32),
         jnp.full((EPAD - E,), NPAD - 1, jnp.int32)])
    r = jnp.concatenate(
        [edge_radial_emb.astype(f32), jnp.zeros((EPAD - E, NB), f32)])
    sh = jnp.concatenate(
        [edge_sh.astype(f32), jnp.zeros((EPAD - E, 1), f32)])
    nf_pad = jnp.concatenate(
        [node_features.astype(f32), jnp.zeros((NPAD - N, D), f32)])

    wdot_bd = jnp.kron(jnp.eye(H, dtype=f32), Wdot.astype(f32))
    wqd = (Wq.astype(f32) @ wdot_bd) * (1.0 / 128.0)
    wout4 = Wout.astype(f32) * 0.25
    wf1s = Wf1.astype(f32) * 0.25
    wf2s = Wf2.astype(f32) * (1.0 / np.sqrt(2 * D))

    sc_gather, sc_aggregate = _build_sc_kernels()
    src2 = src.reshape(EPAD // CHUNK, CHUNK)
    dst2 = dst.reshape(EPAD // CHUNK, CHUNK)
    xs, xd = sc_gather(nf_pad, src2, dst2)
    w1 = jnp.concatenate([Wk1.astype(f32), Wv1.astype(f32)], axis=1)
    b1 = jnp.concatenate([bk1.astype(f32), bv1.astype(f32)]).reshape(1, 2 * HID)
    payload = _edge_stage(
        r, xs, xd, sh, w1, b1,
        Wk2.astype(f32), bk2.astype(f32).reshape(1, D * D),
        bv2.astype(f32).reshape(1, D * D), Wv2.astype(f32),
        wqd, jnp.asarray(_RREP), jnp.asarray(_SSUM), jnp.asarray(_SHF))
    zeros = jnp.zeros((NPAD, PW), f32)
    parts = sc_aggregate(payload, dst2, zeros)
    out = _final_stage(nf_pad, parts, wout4, wf1s, wf2s)
    return out[:N]
